# Initial kernel scaffold; baseline (speedup 1.0000x reference)
import jax
import jax.numpy as jnp
from jax.experimental import pallas as pl


def kernel(state, x, vals_res, rows_res, cols_res, bias_res, vals_in, rows_in, cols_in, bias_in):
    raise NotImplementedError("write your pallas kernel here")


# speedup vs baseline: 1.6670x; 1.6670x over previous
import functools

import jax
import jax.numpy as jnp
from jax import lax
from jax.experimental import pallas as pl
from jax.experimental.pallas import tpu as pltpu
from jax.experimental.pallas import tpu_sc as plsc

N_RES = 16384
N_IN = 1024
BATCH = 256

NC = 2
NS = 16
LANES = 16

NCHUNK = 4
CB = BATCH // NCHUNK
T_ROWS = N_RES + N_IN + 1
K = 128
ZR = 128
ROWS_PER_TILE = N_RES // NS

_DNUMS = lax.GatherDimensionNumbers(
    offset_dims=(), collapsed_slice_dims=(0,), start_index_map=(0,))


def _lane_bcast(v16, l):
    idx = jnp.full((LANES, 1), l, dtype=jnp.int32)
    return lax.gather(v16, idx, _DNUMS, (1,),
                      mode=lax.GatherScatterMode.PROMISE_IN_BOUNDS)


def _make_spmm(n_blk):
    per_tile = n_blk * K
    mesh = plsc.VectorSubcoreMesh(core_axis_name="c", subcore_axis_name="s")

    @functools.partial(
        pl.kernel,
        out_type=jax.ShapeDtypeStruct((NCHUNK, N_RES, CB), jnp.float32),
        mesh=mesh,
        scratch_types=[
            pltpu.VMEM_SHARED((N_RES, CB), jnp.float32),
            pltpu.VMEM((K, CB), jnp.float32),
            pltpu.VMEM((K,), jnp.int32),
            pltpu.VMEM((K,), jnp.int32),
            pltpu.VMEM((K,), jnp.float32),
            pltpu.VMEM((ZR, CB), jnp.float32),
            pltpu.SemaphoreType.DMA,
        ],
        compiler_params=pltpu.CompilerParams(use_tc_tiling_on_sc=False),
    )
    def spmm(t_hbm, cols_hbm, rows_hbm, vals_hbm, y_hbm,
             acc, gbuf, cbuf, rbuf, vbuf, zbuf, sem):
        core = lax.axis_index("c")
        sub = lax.axis_index("s")
        tile_base = sub * per_tile

        zv = jnp.zeros((LANES,), jnp.float32)
        for r in range(ZR):
            for q in range(CB // LANES):
                zbuf[r, pl.ds(q * LANES, LANES)] = zv

        for cc in range(NCHUNK // NC):
            chunk = core * (NCHUNK // NC) + cc
            off = chunk * T_ROWS

            for p in range(ROWS_PER_TILE // ZR):
                pltpu.sync_copy(
                    zbuf, acc.at[pl.ds(sub * ROWS_PER_TILE + p * ZR, ZR)])
            plsc.subcore_barrier()

            def body(b, carry):
                base = tile_base + b * K
                pltpu.sync_copy(cols_hbm.at[pl.ds(base, K)], cbuf)
                pltpu.sync_copy(rows_hbm.at[pl.ds(base, K)], rbuf)
                pltpu.sync_copy(vals_hbm.at[pl.ds(base, K)], vbuf)
                for q in range(K // LANES):
                    sl = pl.ds(q * LANES, LANES)
                    cbuf[sl] = cbuf[sl] + off
                pltpu.async_copy(t_hbm.at[cbuf], gbuf, sem).wait()
                for j0 in range(0, K, LANES):
                    v16 = vbuf[pl.ds(j0, LANES)]
                    for l in range(LANES):
                        bc = _lane_bcast(v16, l)
                        j = j0 + l
                        for q in range(CB // LANES):
                            sl = pl.ds(q * LANES, LANES)
                            gbuf[j, sl] = gbuf[j, sl] * bc
                pltpu.sync_copy(gbuf, acc.at[rbuf], add=True)
                return carry

            lax.fori_loop(0, n_blk, body, 0)
            plsc.subcore_barrier()

            pltpu.sync_copy(
                acc.at[pl.ds(sub * ROWS_PER_TILE, ROWS_PER_TILE)],
                y_hbm.at[chunk, pl.ds(sub * ROWS_PER_TILE, ROWS_PER_TILE)])
            plsc.subcore_barrier()

    return spmm


def _erf_body(y_ref, o_ref):
    z = lax.erf(y_ref[...])
    o_ref[...] = jnp.transpose(z, (0, 2, 1))


RB = 512


def kernel(state, x, vals_res, rows_res, cols_res, bias_res,
           vals_in, rows_in, cols_in, bias_in):
    nnz_res = vals_res.shape[0]
    nnz_in = vals_in.shape[0]
    nnz_tot = nnz_res + nnz_in + N_RES
    n_blk = -(-nnz_tot // (NS * K))
    npad = n_blk * K * NS - nnz_tot

    t = jnp.concatenate(
        [state.T, x.T, jnp.ones((1, BATCH), jnp.float32)], axis=0)
    t4 = t.reshape(T_ROWS, NCHUNK, CB).transpose(1, 0, 2)
    t4 = t4.reshape(NCHUNK * T_ROWS, CB)

    i32 = jnp.int32
    cols = jnp.concatenate([
        cols_res.astype(i32), cols_in.astype(i32) + N_RES,
        jnp.full((N_RES,), N_RES + N_IN, i32), jnp.zeros((npad,), i32)])
    rows = jnp.concatenate([
        rows_res.astype(i32), rows_in.astype(i32),
        jnp.arange(N_RES, dtype=i32), jnp.zeros((npad,), i32)])
    vals = jnp.concatenate([
        vals_res, vals_in, bias_res + bias_in,
        jnp.zeros((npad,), jnp.float32)])

    y4 = _make_spmm(n_blk)(t4, cols, rows, vals)

    out3 = pl.pallas_call(
        _erf_body,
        grid=(N_RES // RB,),
        in_specs=[pl.BlockSpec((NCHUNK, RB, CB), lambda i: (0, i, 0))],
        out_specs=pl.BlockSpec((NCHUNK, CB, RB), lambda i: (0, 0, i)),
        out_shape=jax.ShapeDtypeStruct((NCHUNK, CB, N_RES), jnp.float32),
    )(y4)
    return out3.reshape(BATCH, N_RES)


# speedup vs baseline: 2.9467x; 1.7677x over previous
import functools

import jax
import jax.numpy as jnp
from jax import lax
from jax.experimental import pallas as pl
from jax.experimental.pallas import tpu as pltpu
from jax.experimental.pallas import tpu_sc as plsc

N_RES = 16384
N_IN = 1024
BATCH = 256

NC = 2
NS = 16
LANES = 16

NCHUNK = 4
CB = BATCH // NCHUNK
T_ROWS = N_RES + N_IN + 1
K = 128
ZR = 128
ROWS_PER_TILE = N_RES // NS

_DNUMS = lax.GatherDimensionNumbers(
    offset_dims=(), collapsed_slice_dims=(0,), start_index_map=(0,))


def _lane_bcast(v16, l):
    idx = jnp.full((LANES, 1), l, dtype=jnp.int32)
    return lax.gather(v16, idx, _DNUMS, (1,),
                      mode=lax.GatherScatterMode.PROMISE_IN_BOUNDS)


def _make_spmm(n_blk):
    per_tile = n_blk * K
    mesh = plsc.VectorSubcoreMesh(core_axis_name="c", subcore_axis_name="s")

    @functools.partial(
        pl.kernel,
        out_type=jax.ShapeDtypeStruct((NCHUNK, N_RES, CB), jnp.float32),
        mesh=mesh,
        scratch_types=[
            pltpu.VMEM_SHARED((N_RES, CB), jnp.float32),
            pltpu.VMEM((K, CB), jnp.float32),
            pltpu.VMEM((K, CB), jnp.float32),
            pltpu.VMEM((K,), jnp.int32),
            pltpu.VMEM((K,), jnp.int32),
            pltpu.VMEM((K,), jnp.int32),
            pltpu.VMEM((K,), jnp.int32),
            pltpu.VMEM((K,), jnp.float32),
            pltpu.VMEM((K,), jnp.float32),
            pltpu.VMEM((ZR, CB), jnp.float32),
            pltpu.SemaphoreType.DMA,
            pltpu.SemaphoreType.DMA,
            pltpu.SemaphoreType.DMA,
            pltpu.SemaphoreType.DMA,
            pltpu.SemaphoreType.DMA,
        ],
        compiler_params=pltpu.CompilerParams(use_tc_tiling_on_sc=False),
    )
    def spmm(t_hbm, cols_hbm, rows_hbm, vals_hbm, y_hbm,
             acc, g0, g1, cb0, cb1, rb0, rb1, vb0, vb1, zbuf,
             si0, si1, sg0, sg1, ss):
        core = lax.axis_index("c")
        sub = lax.axis_index("s")
        tile_base = sub * per_tile
        G = (g0, g1)
        CBF = (cb0, cb1)
        RBF = (rb0, rb1)
        VBF = (vb0, vb1)
        SI = (si0, si1)
        SG = (sg0, sg1)

        def idx_start(i, p, chunk):
            base = tile_base + i * K
            d1 = pltpu.async_copy(
                cols_hbm.at[chunk, pl.ds(base, K)], CBF[p], SI[p])
            d2 = pltpu.async_copy(rows_hbm.at[pl.ds(base, K)], RBF[p], SI[p])
            d3 = pltpu.async_copy(vals_hbm.at[pl.ds(base, K)], VBF[p], SI[p])
            return d1, d2, d3

        def idx_wait(p, chunk):
            for d in idx_make(p, chunk):
                d.wait()

        def idx_make(p, chunk):
            return (
                pltpu.make_async_copy(
                    cols_hbm.at[chunk, pl.ds(0, K)], CBF[p], SI[p]),
                pltpu.make_async_copy(rows_hbm.at[pl.ds(0, K)], RBF[p], SI[p]),
                pltpu.make_async_copy(vals_hbm.at[pl.ds(0, K)], VBF[p], SI[p]),
            )

        def gather_start(p):
            return pltpu.async_copy(t_hbm.at[CBF[p]], G[p], SG[p])

        def gather_wait(p):
            pltpu.make_async_copy(t_hbm.at[CBF[p]], G[p], SG[p]).wait()

        def scale(p):
            gbuf = G[p]
            vbuf = VBF[p]
            for j0 in range(0, K, LANES):
                v16 = vbuf[pl.ds(j0, LANES)]
                for l in range(LANES):
                    bc = _lane_bcast(v16, l)
                    j = j0 + l
                    for q in range(CB // LANES):
                        sl = pl.ds(q * LANES, LANES)
                        gbuf[j, sl] = gbuf[j, sl] * bc

        zv = jnp.zeros((LANES,), jnp.float32)
        for r in range(ZR):
            for q in range(CB // LANES):
                zbuf[r, pl.ds(q * LANES, LANES)] = zv

        def chunk_body(cc, carry):
            chunk = core * (NCHUNK // NC) + cc

            for p in range(ROWS_PER_TILE // ZR):
                pltpu.sync_copy(
                    zbuf, acc.at[pl.ds(sub * ROWS_PER_TILE + p * ZR, ZR)])
            plsc.subcore_barrier()

            idx_start(0, 0, chunk)
            idx_start(1, 1, chunk)
            idx_wait(0, chunk)
            gather_start(0)

            def body(q, carry):
                for r in range(2):
                    i = 2 * q + r
                    cur, oth = r, 1 - r
                    idx_wait(oth, chunk)
                    gather_start(oth)
                    gather_wait(cur)
                    scale(cur)
                    d = pltpu.async_copy(
                        G[cur], acc.at[RBF[cur]], ss, add=True)
                    d.wait()
                    idx_start(i + 2, cur, chunk)
                return carry

            lax.fori_loop(0, n_blk // 2, body, 0)
            gather_wait(n_blk % 2)
            idx_wait((n_blk + 1) % 2, chunk)
            plsc.subcore_barrier()

            pltpu.sync_copy(
                acc.at[pl.ds(sub * ROWS_PER_TILE, ROWS_PER_TILE)],
                y_hbm.at[chunk, pl.ds(sub * ROWS_PER_TILE, ROWS_PER_TILE)])
            plsc.subcore_barrier()
            return carry

        lax.fori_loop(0, NCHUNK // NC, chunk_body, 0)

    return spmm


def _erf_body(y_ref, o_ref):
    z = lax.erf(y_ref[...])
    o_ref[...] = jnp.transpose(z, (0, 2, 1))


RB = 512


def kernel(state, x, vals_res, rows_res, cols_res, bias_res,
           vals_in, rows_in, cols_in, bias_in):
    nnz_res = vals_res.shape[0]
    nnz_in = vals_in.shape[0]
    nnz_tot = nnz_res + nnz_in + N_RES
    n_blk = -(-nnz_tot // (NS * K))
    n_blk += n_blk % 2
    npad = n_blk * K * NS + 2 * K - nnz_tot

    t = jnp.concatenate(
        [state.T, x.T, jnp.ones((1, BATCH), jnp.float32)], axis=0)
    t4 = t.reshape(T_ROWS, NCHUNK, CB).transpose(1, 0, 2)
    t4 = t4.reshape(NCHUNK * T_ROWS, CB)

    i32 = jnp.int32
    cols = jnp.concatenate([
        cols_res.astype(i32), cols_in.astype(i32) + N_RES,
        jnp.full((N_RES,), N_RES + N_IN, i32), jnp.zeros((npad,), i32)])
    rows = jnp.concatenate([
        rows_res.astype(i32), rows_in.astype(i32),
        jnp.arange(N_RES, dtype=i32), jnp.zeros((npad,), i32)])
    vals = jnp.concatenate([
        vals_res, vals_in, bias_res + bias_in,
        jnp.zeros((npad,), jnp.float32)])
    cols_c = cols[None, :] + (jnp.arange(NCHUNK, dtype=i32) * T_ROWS)[:, None]

    y4 = _make_spmm(n_blk)(t4, cols_c, rows, vals)

    out3 = pl.pallas_call(
        _erf_body,
        grid=(N_RES // RB,),
        in_specs=[pl.BlockSpec((NCHUNK, RB, CB), lambda i: (0, i, 0))],
        out_specs=pl.BlockSpec((NCHUNK, CB, RB), lambda i: (0, 0, i)),
        out_shape=jax.ShapeDtypeStruct((NCHUNK, CB, N_RES), jnp.float32),
    )(y4)
    return out3.reshape(BATCH, N_RES)


# speedup vs baseline: 4.0511x; 1.3748x over previous
import functools

import jax
import jax.numpy as jnp
from jax import lax
from jax.experimental import pallas as pl
from jax.experimental.pallas import tpu as pltpu
from jax.experimental.pallas import tpu_sc as plsc

N_RES = 16384
N_IN = 1024
BATCH = 256

NC = 2
NS = 16
LANES = 16

NCHUNK = 8
CB = BATCH // NCHUNK
T_ROWS = N_RES + N_IN + 1
T_PAD = 17424
SLAB_PT = T_PAD // NS
K = 128
ZR = 128
ROWS_PER_TILE = N_RES // NS

_DNUMS = lax.GatherDimensionNumbers(
    offset_dims=(), collapsed_slice_dims=(0,), start_index_map=(0,))


def _lane_bcast(v16, l):
    idx = jnp.full((LANES, 1), l, dtype=jnp.int32)
    return lax.gather(v16, idx, _DNUMS, (1,),
                      mode=lax.GatherScatterMode.PROMISE_IN_BOUNDS)


def _make_spmm(n_blk):
    per_tile = n_blk * K
    mesh = plsc.VectorSubcoreMesh(core_axis_name="c", subcore_axis_name="s")

    @functools.partial(
        pl.kernel,
        out_type=jax.ShapeDtypeStruct((NCHUNK, N_RES, CB), jnp.float32),
        mesh=mesh,
        scratch_types=[
            pltpu.VMEM_SHARED((N_RES, CB), jnp.float32),
            pltpu.VMEM_SHARED((T_PAD, CB), jnp.float32),
            pltpu.VMEM((K, CB), jnp.float32),
            pltpu.VMEM((K, CB), jnp.float32),
            pltpu.VMEM((K,), jnp.int32),
            pltpu.VMEM((K,), jnp.int32),
            pltpu.VMEM((K,), jnp.int32),
            pltpu.VMEM((K,), jnp.int32),
            pltpu.VMEM((K,), jnp.float32),
            pltpu.VMEM((K,), jnp.float32),
            pltpu.VMEM((ZR, CB), jnp.float32),
            pltpu.SemaphoreType.DMA,
            pltpu.SemaphoreType.DMA,
            pltpu.SemaphoreType.DMA,
            pltpu.SemaphoreType.DMA,
            pltpu.SemaphoreType.DMA,
        ],
        compiler_params=pltpu.CompilerParams(use_tc_tiling_on_sc=False),
    )
    def spmm(t_hbm, cols_hbm, rows_hbm, vals_hbm, y_hbm,
             acc, tslab, g0, g1, cb0, cb1, rb0, rb1, vb0, vb1, zbuf,
             si0, si1, sg0, sg1, ss):
        core = lax.axis_index("c")
        sub = lax.axis_index("s")
        tile_base = sub * per_tile
        G = (g0, g1)
        CBF = (cb0, cb1)
        RBF = (rb0, rb1)
        VBF = (vb0, vb1)
        SI = (si0, si1)
        SG = (sg0, sg1)

        def idx_start(i, p):
            base = tile_base + i * K
            d1 = pltpu.async_copy(cols_hbm.at[pl.ds(base, K)], CBF[p], SI[p])
            d2 = pltpu.async_copy(rows_hbm.at[pl.ds(base, K)], RBF[p], SI[p])
            d3 = pltpu.async_copy(vals_hbm.at[pl.ds(base, K)], VBF[p], SI[p])
            return d1, d2, d3

        def idx_wait(p):
            for d in idx_make(p):
                d.wait()

        def idx_make(p):
            return (
                pltpu.make_async_copy(
                    cols_hbm.at[pl.ds(0, K)], CBF[p], SI[p]),
                pltpu.make_async_copy(rows_hbm.at[pl.ds(0, K)], RBF[p], SI[p]),
                pltpu.make_async_copy(vals_hbm.at[pl.ds(0, K)], VBF[p], SI[p]),
            )

        def gather_start(p):
            return pltpu.async_copy(tslab.at[CBF[p]], G[p], SG[p])

        def gather_wait(p):
            pltpu.make_async_copy(tslab.at[CBF[p]], G[p], SG[p]).wait()

        def scale(p):
            gbuf = G[p]
            vbuf = VBF[p]
            for j0 in range(0, K, LANES):
                v16 = vbuf[pl.ds(j0, LANES)]
                for l in range(LANES):
                    bc = _lane_bcast(v16, l)
                    j = j0 + l
                    for q in range(CB // LANES):
                        sl = pl.ds(q * LANES, LANES)
                        gbuf[j, sl] = gbuf[j, sl] * bc

        zv = jnp.zeros((LANES,), jnp.float32)
        for r in range(ZR):
            for q in range(CB // LANES):
                zbuf[r, pl.ds(q * LANES, LANES)] = zv

        def chunk_body(cc, carry):
            chunk = core * (NCHUNK // NC) + cc

            pltpu.sync_copy(
                t_hbm.at[pl.ds(chunk * T_PAD + sub * SLAB_PT, SLAB_PT)],
                tslab.at[pl.ds(sub * SLAB_PT, SLAB_PT)])
            for p in range(ROWS_PER_TILE // ZR):
                pltpu.sync_copy(
                    zbuf, acc.at[pl.ds(sub * ROWS_PER_TILE + p * ZR, ZR)])
            plsc.subcore_barrier()

            idx_start(0, 0)
            idx_start(1, 1)
            idx_wait(0)
            gather_start(0)

            def body(q, carry):
                for r in range(2):
                    i = 2 * q + r
                    cur, oth = r, 1 - r
                    idx_wait(oth)
                    gather_start(oth)
                    gather_wait(cur)
                    scale(cur)
                    d = pltpu.async_copy(
                        G[cur], acc.at[RBF[cur]], ss, add=True)
                    d.wait()
                    idx_start(i + 2, cur)
                return carry

            lax.fori_loop(0, n_blk // 2, body, 0)
            gather_wait(n_blk % 2)
            idx_wait((n_blk + 1) % 2)
            plsc.subcore_barrier()

            pltpu.sync_copy(
                acc.at[pl.ds(sub * ROWS_PER_TILE, ROWS_PER_TILE)],
                y_hbm.at[chunk, pl.ds(sub * ROWS_PER_TILE, ROWS_PER_TILE)])
            plsc.subcore_barrier()
            return carry

        lax.fori_loop(0, NCHUNK // NC, chunk_body, 0)

    return spmm


def _erf_body(y_ref, o_ref):
    z = lax.erf(y_ref[...])
    o_ref[...] = jnp.transpose(z, (0, 2, 1))


RB = 512


def kernel(state, x, vals_res, rows_res, cols_res, bias_res,
           vals_in, rows_in, cols_in, bias_in):
    nnz_res = vals_res.shape[0]
    nnz_in = vals_in.shape[0]
    nnz_tot = nnz_res + nnz_in + N_RES
    n_blk = -(-nnz_tot // (NS * K))
    n_blk += n_blk % 2
    npad = n_blk * K * NS + 2 * K - nnz_tot

    t = jnp.concatenate(
        [state.T, x.T, jnp.ones((1, BATCH), jnp.float32),
         jnp.zeros((T_PAD - T_ROWS, BATCH), jnp.float32)], axis=0)
    t4 = t.reshape(T_PAD, NCHUNK, CB).transpose(1, 0, 2)
    t4 = t4.reshape(NCHUNK * T_PAD, CB)

    i32 = jnp.int32
    cols = jnp.concatenate([
        cols_res.astype(i32), cols_in.astype(i32) + N_RES,
        jnp.full((N_RES,), N_RES + N_IN, i32), jnp.zeros((npad,), i32)])
    rows = jnp.concatenate([
        rows_res.astype(i32), rows_in.astype(i32),
        jnp.arange(N_RES, dtype=i32), jnp.zeros((npad,), i32)])
    vals = jnp.concatenate([
        vals_res, vals_in, bias_res + bias_in,
        jnp.zeros((npad,), jnp.float32)])

    y4 = _make_spmm(n_blk)(t4, cols, rows, vals)

    out3 = pl.pallas_call(
        _erf_body,
        grid=(N_RES // RB,),
        in_specs=[pl.BlockSpec((NCHUNK, RB, CB), lambda i: (0, i, 0))],
        out_specs=pl.BlockSpec((NCHUNK, CB, RB), lambda i: (0, 0, i)),
        out_shape=jax.ShapeDtypeStruct((NCHUNK, CB, N_RES), jnp.float32),
    )(y4)
    return out3.reshape(BATCH, N_RES)


# speedup vs baseline: 5.5248x; 1.3638x over previous
import functools

import jax
import jax.numpy as jnp
from jax import lax
from jax.experimental import pallas as pl
from jax.experimental.pallas import tpu as pltpu
from jax.experimental.pallas import tpu_sc as plsc

N_RES = 16384
N_IN = 1024
BATCH = 256

NC = 2
NS = 16
LANES = 16

NCHUNK = 4
CB = BATCH // NCHUNK
T_ROWS = N_RES + N_IN + 1
T_PAD = 17424
SLAB_PT = T_PAD // NS
K = 128
ZR = 64
ROWS_PER_TILE = N_RES // NS

_DNUMS = lax.GatherDimensionNumbers(
    offset_dims=(), collapsed_slice_dims=(0,), start_index_map=(0,))

import numpy as _np
_UNPACK_ORDER = _np.concatenate(
    [_np.arange(0, 32, 2), _np.arange(1, 32, 2),
     32 + _np.arange(0, 32, 2), 32 + _np.arange(1, 32, 2)])
_COL_PERM = _np.concatenate(
    [c * CB + _np.argsort(_UNPACK_ORDER) for c in range(NCHUNK)])


def _lane_bcast(v16, l):
    idx = jnp.full((LANES, 1), l, dtype=jnp.int32)
    return lax.gather(v16, idx, _DNUMS, (1,),
                      mode=lax.GatherScatterMode.PROMISE_IN_BOUNDS)


def _make_spmm(n_blk):
    per_tile = n_blk * K
    mesh = plsc.VectorSubcoreMesh(core_axis_name="c", subcore_axis_name="s")

    @functools.partial(
        pl.kernel,
        out_type=jax.ShapeDtypeStruct((NCHUNK, N_RES, CB), jnp.float32),
        mesh=mesh,
        scratch_types=[
            pltpu.VMEM_SHARED((N_RES, CB), jnp.float32),
            pltpu.VMEM_SHARED((T_PAD, CB), jnp.bfloat16),
            pltpu.VMEM((K, CB), jnp.bfloat16),
            pltpu.VMEM((K, CB), jnp.bfloat16),
            pltpu.VMEM((K, CB), jnp.float32),
            pltpu.VMEM((K,), jnp.int32),
            pltpu.VMEM((K,), jnp.int32),
            pltpu.VMEM((K,), jnp.int32),
            pltpu.VMEM((K,), jnp.int32),
            pltpu.VMEM((K,), jnp.float32),
            pltpu.VMEM((K,), jnp.float32),
            pltpu.VMEM((ZR, CB), jnp.float32),
            pltpu.SemaphoreType.DMA,
            pltpu.SemaphoreType.DMA,
            pltpu.SemaphoreType.DMA,
            pltpu.SemaphoreType.DMA,
            pltpu.SemaphoreType.DMA,
        ],
        compiler_params=pltpu.CompilerParams(
            use_tc_tiling_on_sc=False, needs_layout_passes=False),
    )
    def spmm(t_hbm, cols_hbm, rows_hbm, vals_hbm, y_hbm,
             acc, tslab, h0, h1, gb, cb0, cb1, rb0, rb1, vb0, vb1, zbuf,
             si0, si1, sg0, sg1, ss):
        core = lax.axis_index("c")
        sub = lax.axis_index("s")
        tile_base = sub * per_tile
        H = (h0, h1)
        CBF = (cb0, cb1)
        RBF = (rb0, rb1)
        VBF = (vb0, vb1)
        SI = (si0, si1)
        SG = (sg0, sg1)

        def idx_start(i, p):
            base = tile_base + i * K
            d1 = pltpu.async_copy(cols_hbm.at[pl.ds(base, K)], CBF[p], SI[p])
            d2 = pltpu.async_copy(rows_hbm.at[pl.ds(base, K)], RBF[p], SI[p])
            d3 = pltpu.async_copy(vals_hbm.at[pl.ds(base, K)], VBF[p], SI[p])
            return d1, d2, d3

        def idx_wait(p):
            for d in idx_make(p):
                d.wait()

        def idx_make(p):
            return (
                pltpu.make_async_copy(
                    cols_hbm.at[pl.ds(0, K)], CBF[p], SI[p]),
                pltpu.make_async_copy(rows_hbm.at[pl.ds(0, K)], RBF[p], SI[p]),
                pltpu.make_async_copy(vals_hbm.at[pl.ds(0, K)], VBF[p], SI[p]),
            )

        def gather_start(p):
            return pltpu.async_copy(tslab.at[CBF[p]], H[p], SG[p])

        def gather_wait(p):
            pltpu.make_async_copy(tslab.at[CBF[p]], H[p], SG[p]).wait()

        def scale(p):
            hbuf = H[p]
            gbuf = gb
            vbuf = VBF[p]
            for j0 in range(0, K, LANES):
                v16 = vbuf[pl.ds(j0, LANES)]
                for l in range(LANES):
                    bc = _lane_bcast(v16, l)
                    j = j0 + l
                    for q in range(CB // 32):
                        h32 = hbuf[j, pl.ds(q * 32, 32)]
                        fa, fb = plsc.unpack(
                            h32, format=plsc.PackFormat.INTERLEAVED)
                        gbuf[j, pl.ds(q * 32, LANES)] = fa * bc
                        gbuf[j, pl.ds(q * 32 + LANES, LANES)] = fb * bc

        zv = jnp.zeros((LANES,), jnp.float32)
        for r in range(ZR):
            for q in range(CB // LANES):
                zbuf[r, pl.ds(q * LANES, LANES)] = zv

        def chunk_body(cc, carry):
            chunk = core * (NCHUNK // NC) + cc

            pltpu.sync_copy(
                t_hbm.at[pl.ds(chunk * T_PAD + sub * SLAB_PT, SLAB_PT)],
                tslab.at[pl.ds(sub * SLAB_PT, SLAB_PT)])
            for p in range(ROWS_PER_TILE // ZR):
                pltpu.sync_copy(
                    zbuf, acc.at[pl.ds(sub * ROWS_PER_TILE + p * ZR, ZR)])
            plsc.subcore_barrier()

            idx_start(0, 0)
            idx_start(1, 1)
            idx_wait(0)
            gather_start(0)

            def body(q, carry):
                for r in range(2):
                    i = 2 * q + r
                    cur, oth = r, 1 - r
                    idx_wait(oth)
                    gather_start(oth)
                    gather_wait(cur)
                    scale(cur)
                    d = pltpu.async_copy(
                        gb, acc.at[RBF[cur]], ss, add=True)
                    d.wait()
                    idx_start(i + 2, cur)
                return carry

            lax.fori_loop(0, n_blk // 2, body, 0)
            gather_wait(n_blk % 2)
            idx_wait((n_blk + 1) % 2)
            plsc.subcore_barrier()

            pltpu.sync_copy(
                acc.at[pl.ds(sub * ROWS_PER_TILE, ROWS_PER_TILE)],
                y_hbm.at[chunk, pl.ds(sub * ROWS_PER_TILE, ROWS_PER_TILE)])
            plsc.subcore_barrier()
            return carry

        lax.fori_loop(0, NCHUNK // NC, chunk_body, 0)

    return spmm


def _erf_body(y_ref, o_ref):
    z = lax.erf(y_ref[...])
    o_ref[...] = jnp.transpose(z, (0, 2, 1))


RB = 512


def kernel(state, x, vals_res, rows_res, cols_res, bias_res,
           vals_in, rows_in, cols_in, bias_in):
    nnz_res = vals_res.shape[0]
    nnz_in = vals_in.shape[0]
    nnz_tot = nnz_res + nnz_in + N_RES
    n_blk = -(-nnz_tot // (NS * K))
    n_blk += n_blk % 2
    npad = n_blk * K * NS + 2 * K - nnz_tot

    t = jnp.concatenate(
        [state.T, x.T, jnp.ones((1, BATCH), jnp.float32),
         jnp.zeros((T_PAD - T_ROWS, BATCH), jnp.float32)], axis=0)
    t = t[:, _COL_PERM].astype(jnp.bfloat16)
    t4 = t.reshape(T_PAD, NCHUNK, CB).transpose(1, 0, 2)
    t4 = t4.reshape(NCHUNK * T_PAD, CB)

    i32 = jnp.int32
    cols = jnp.concatenate([
        cols_res.astype(i32), cols_in.astype(i32) + N_RES,
        jnp.full((N_RES,), N_RES + N_IN, i32), jnp.zeros((npad,), i32)])
    rows = jnp.concatenate([
        rows_res.astype(i32), rows_in.astype(i32),
        jnp.arange(N_RES, dtype=i32), jnp.zeros((npad,), i32)])
    vals = jnp.concatenate([
        vals_res, vals_in, bias_res + bias_in,
        jnp.zeros((npad,), jnp.float32)])

    y4 = _make_spmm(n_blk)(t4, cols, rows, vals)

    out3 = pl.pallas_call(
        _erf_body,
        grid=(N_RES // RB,),
        in_specs=[pl.BlockSpec((NCHUNK, RB, CB), lambda i: (0, i, 0))],
        out_specs=pl.BlockSpec((NCHUNK, CB, RB), lambda i: (0, 0, i)),
        out_shape=jax.ShapeDtypeStruct((NCHUNK, CB, N_RES), jnp.float32),
    )(y4)
    return out3.reshape(BATCH, N_RES)


# speedup vs baseline: 6.3233x; 1.1445x over previous
import functools

import jax
import jax.numpy as jnp
from jax import lax
from jax.experimental import pallas as pl
from jax.experimental.pallas import tpu as pltpu
from jax.experimental.pallas import tpu_sc as plsc

N_RES = 16384
N_IN = 1024
BATCH = 256

NC = 2
NS = 16
LANES = 16

NCHUNK = 4
CB = BATCH // NCHUNK
T_ROWS = N_RES + N_IN + 1
T_PAD = 17424
SLAB_PT = T_PAD // NS
K = 128
ZR = 64
ROWS_PER_TILE = N_RES // NS

_DNUMS = lax.GatherDimensionNumbers(
    offset_dims=(), collapsed_slice_dims=(0,), start_index_map=(0,))

import numpy as _np
_UNPACK_ORDER = _np.concatenate(
    [_np.arange(0, 32, 2), _np.arange(1, 32, 2),
     32 + _np.arange(0, 32, 2), 32 + _np.arange(1, 32, 2)])
_COL_PERM = _np.concatenate(
    [c * CB + _np.argsort(_UNPACK_ORDER) for c in range(NCHUNK)])


def _lane_bcast(v16, l):
    idx = jnp.full((LANES, 1), l, dtype=jnp.int32)
    return lax.gather(v16, idx, _DNUMS, (1,),
                      mode=lax.GatherScatterMode.PROMISE_IN_BOUNDS)


def _make_spmm(n_blk):
    per_tile = n_blk * K
    mesh = plsc.VectorSubcoreMesh(core_axis_name="c", subcore_axis_name="s")

    @functools.partial(
        pl.kernel,
        out_type=jax.ShapeDtypeStruct((NCHUNK, N_RES, CB), jnp.float32),
        mesh=mesh,
        scratch_types=[
            pltpu.VMEM_SHARED((N_RES, CB), jnp.float32),
            pltpu.VMEM_SHARED((T_PAD, CB), jnp.bfloat16),
            pltpu.VMEM((K, CB), jnp.bfloat16),
            pltpu.VMEM((K, CB), jnp.bfloat16),
            pltpu.VMEM((K, CB), jnp.float32),
            pltpu.VMEM((K, CB), jnp.float32),
            pltpu.VMEM((K,), jnp.int32),
            pltpu.VMEM((K,), jnp.int32),
            pltpu.VMEM((K,), jnp.int32),
            pltpu.VMEM((K,), jnp.int32),
            pltpu.VMEM((K,), jnp.float32),
            pltpu.VMEM((K,), jnp.float32),
            pltpu.VMEM((K,), jnp.int32),
            pltpu.VMEM((K,), jnp.int32),
            pltpu.VMEM((ZR, CB), jnp.float32),
            pltpu.SemaphoreType.DMA,
            pltpu.SemaphoreType.DMA,
            pltpu.SemaphoreType.DMA,
            pltpu.SemaphoreType.DMA,
            pltpu.SemaphoreType.DMA,
            pltpu.SemaphoreType.DMA,
        ],
        compiler_params=pltpu.CompilerParams(
            use_tc_tiling_on_sc=False, needs_layout_passes=False),
    )
    def spmm(t_hbm, cols_hbm, rows_hbm, vals_hbm, y_hbm,
             acc, tslab, h0, h1, g0, g1, cb0, cb1, rb0, rb1, vb0, vb1,
             sr0, sr1, zbuf, si0, si1, sg0, sg1, ss0, ss1):
        core = lax.axis_index("c")
        sub = lax.axis_index("s")
        tile_base = sub * per_tile
        H = (h0, h1)
        G = (g0, g1)
        SRB = (sr0, sr1)
        SS = (ss0, ss1)
        CBF = (cb0, cb1)
        RBF = (rb0, rb1)
        VBF = (vb0, vb1)
        SI = (si0, si1)
        SG = (sg0, sg1)

        def idx_start(i, p):
            base = tile_base + i * K
            d1 = pltpu.async_copy(cols_hbm.at[pl.ds(base, K)], CBF[p], SI[p])
            d2 = pltpu.async_copy(rows_hbm.at[pl.ds(base, K)], RBF[p], SI[p])
            d3 = pltpu.async_copy(vals_hbm.at[pl.ds(base, K)], VBF[p], SI[p])
            return d1, d2, d3

        def idx_wait(p):
            for d in idx_make(p):
                d.wait()

        def idx_make(p):
            return (
                pltpu.make_async_copy(
                    cols_hbm.at[pl.ds(0, K)], CBF[p], SI[p]),
                pltpu.make_async_copy(rows_hbm.at[pl.ds(0, K)], RBF[p], SI[p]),
                pltpu.make_async_copy(vals_hbm.at[pl.ds(0, K)], VBF[p], SI[p]),
            )

        def gather_start(p):
            return pltpu.async_copy(tslab.at[CBF[p]], H[p], SG[p])

        def gather_wait(p):
            pltpu.make_async_copy(tslab.at[CBF[p]], H[p], SG[p]).wait()

        def scat_wait(p):
            pltpu.make_async_copy(
                G[p], acc.at[SRB[p]], SS[p]).wait()

        def scale(p):
            hbuf = H[p]
            gbuf = G[p]
            vbuf = VBF[p]
            for j0 in range(0, K, LANES):
                v16 = vbuf[pl.ds(j0, LANES)]
                for l in range(LANES):
                    bc = _lane_bcast(v16, l)
                    j = j0 + l
                    for q in range(CB // 32):
                        h32 = hbuf[j, pl.ds(q * 32, 32)]
                        fa, fb = plsc.unpack(
                            h32, format=plsc.PackFormat.INTERLEAVED)
                        gbuf[j, pl.ds(q * 32, LANES)] = fa * bc
                        gbuf[j, pl.ds(q * 32 + LANES, LANES)] = fb * bc

        zv = jnp.zeros((LANES,), jnp.float32)
        for r in range(ZR):
            for q in range(CB // LANES):
                zbuf[r, pl.ds(q * LANES, LANES)] = zv

        def chunk_body(cc, carry):
            chunk = core * (NCHUNK // NC) + cc

            pltpu.sync_copy(
                t_hbm.at[pl.ds(chunk * T_PAD + sub * SLAB_PT, SLAB_PT)],
                tslab.at[pl.ds(sub * SLAB_PT, SLAB_PT)])
            for p in range(ROWS_PER_TILE // ZR):
                pltpu.sync_copy(
                    zbuf, acc.at[pl.ds(sub * ROWS_PER_TILE + p * ZR, ZR)])
            plsc.subcore_barrier()

            idx_start(0, 0)
            idx_start(1, 1)
            idx_wait(0)
            gather_start(0)

            def body(q, carry):
                for r in range(2):
                    i = 2 * q + r
                    cur, oth = r, 1 - r
                    idx_wait(oth)
                    gather_start(oth)
                    gather_wait(cur)
                    pl.when(q >= 1)(functools.partial(scat_wait, cur))
                    scale(cur)
                    for u in range(K // LANES):
                        sl = pl.ds(u * LANES, LANES)
                        SRB[cur][sl] = RBF[cur][sl]
                    pltpu.async_copy(
                        G[cur], acc.at[SRB[cur]], SS[cur], add=True)
                    idx_start(i + 2, cur)
                return carry

            lax.fori_loop(0, n_blk // 2, body, 0)
            scat_wait(0)
            scat_wait(1)
            gather_wait(n_blk % 2)
            idx_wait((n_blk + 1) % 2)
            plsc.subcore_barrier()

            pltpu.sync_copy(
                acc.at[pl.ds(sub * ROWS_PER_TILE, ROWS_PER_TILE)],
                y_hbm.at[chunk, pl.ds(sub * ROWS_PER_TILE, ROWS_PER_TILE)])
            plsc.subcore_barrier()
            return carry

        lax.fori_loop(0, NCHUNK // NC, chunk_body, 0)

    return spmm


def _erf_body(y_ref, o_ref):
    z = lax.erf(y_ref[...])
    o_ref[...] = jnp.transpose(z, (0, 2, 1))


RB = 512


def kernel(state, x, vals_res, rows_res, cols_res, bias_res,
           vals_in, rows_in, cols_in, bias_in):
    nnz_res = vals_res.shape[0]
    nnz_in = vals_in.shape[0]
    nnz_tot = nnz_res + nnz_in + N_RES
    n_blk = -(-nnz_tot // (NS * K))
    n_blk += n_blk % 2
    npad = n_blk * K * NS + 2 * K - nnz_tot

    t = jnp.concatenate(
        [state.T, x.T, jnp.ones((1, BATCH), jnp.float32),
         jnp.zeros((T_PAD - T_ROWS, BATCH), jnp.float32)], axis=0)
    t = t[:, _COL_PERM].astype(jnp.bfloat16)
    t4 = t.reshape(T_PAD, NCHUNK, CB).transpose(1, 0, 2)
    t4 = t4.reshape(NCHUNK * T_PAD, CB)

    i32 = jnp.int32
    cols = jnp.concatenate([
        cols_res.astype(i32), cols_in.astype(i32) + N_RES,
        jnp.full((N_RES,), N_RES + N_IN, i32), jnp.zeros((npad,), i32)])
    rows = jnp.concatenate([
        rows_res.astype(i32), rows_in.astype(i32),
        jnp.arange(N_RES, dtype=i32), jnp.zeros((npad,), i32)])
    vals = jnp.concatenate([
        vals_res, vals_in, bias_res + bias_in,
        jnp.zeros((npad,), jnp.float32)])

    y4 = _make_spmm(n_blk)(t4, cols, rows, vals)

    out3 = pl.pallas_call(
        _erf_body,
        grid=(N_RES // RB,),
        in_specs=[pl.BlockSpec((NCHUNK, RB, CB), lambda i: (0, i, 0))],
        out_specs=pl.BlockSpec((NCHUNK, CB, RB), lambda i: (0, 0, i)),
        out_shape=jax.ShapeDtypeStruct((NCHUNK, CB, N_RES), jnp.float32),
    )(y4)
    return out3.reshape(BATCH, N_RES)


# speedup vs baseline: 6.6090x; 1.0452x over previous
import functools

import jax
import jax.numpy as jnp
from jax import lax
from jax.experimental import pallas as pl
from jax.experimental.pallas import tpu as pltpu
from jax.experimental.pallas import tpu_sc as plsc

N_RES = 16384
N_IN = 1024
BATCH = 256

NC = 2
NS = 16
LANES = 16

NCHUNK = 4
CB = BATCH // NCHUNK
T_ROWS = N_RES + N_IN + 1
T_PAD = 17424
SLAB_PT = T_PAD // NS
K = 128
ZR = 64
ROWS_PER_TILE = N_RES // NS

_DNUMS = lax.GatherDimensionNumbers(
    offset_dims=(), collapsed_slice_dims=(0,), start_index_map=(0,))

import numpy as _np
_UNPACK_ORDER = _np.concatenate(
    [_np.arange(0, 32, 2), _np.arange(1, 32, 2),
     32 + _np.arange(0, 32, 2), 32 + _np.arange(1, 32, 2)])
_COL_PERM = _np.concatenate(
    [c * CB + _np.argsort(_UNPACK_ORDER) for c in range(NCHUNK)])


def _lane_bcast(v16, l):
    idx = jnp.full((LANES, 1), l, dtype=jnp.int32)
    return lax.gather(v16, idx, _DNUMS, (1,),
                      mode=lax.GatherScatterMode.PROMISE_IN_BOUNDS)


def _make_spmm(n_blk):
    per_tile = n_blk * K
    mesh = plsc.VectorSubcoreMesh(core_axis_name="c", subcore_axis_name="s")

    @functools.partial(
        pl.kernel,
        out_type=jax.ShapeDtypeStruct((NCHUNK, N_RES, CB), jnp.float32),
        mesh=mesh,
        scratch_types=[
            pltpu.VMEM_SHARED((N_RES, CB), jnp.float32),
            pltpu.VMEM_SHARED((T_PAD, CB), jnp.bfloat16),
            pltpu.VMEM((K, CB), jnp.bfloat16),
            pltpu.VMEM((K, CB), jnp.bfloat16),
            pltpu.VMEM((K, CB), jnp.float32),
            pltpu.VMEM((K, CB), jnp.float32),
            pltpu.VMEM((K,), jnp.int32),
            pltpu.VMEM((K,), jnp.int32),
            pltpu.VMEM((K,), jnp.int32),
            pltpu.VMEM((K,), jnp.int32),
            pltpu.VMEM((K,), jnp.float32),
            pltpu.VMEM((K,), jnp.float32),
            pltpu.VMEM((K,), jnp.int32),
            pltpu.VMEM((K,), jnp.int32),
            pltpu.VMEM((ZR, CB), jnp.float32),
            pltpu.SemaphoreType.DMA,
            pltpu.SemaphoreType.DMA,
            pltpu.SemaphoreType.DMA,
            pltpu.SemaphoreType.DMA,
            pltpu.SemaphoreType.DMA,
            pltpu.SemaphoreType.DMA,
        ],
        compiler_params=pltpu.CompilerParams(
            use_tc_tiling_on_sc=False, needs_layout_passes=False),
    )
    def spmm(t_hbm, cols_hbm, rows_hbm, vals_hbm, y_hbm,
             acc, tslab, h0, h1, g0, g1, cb0, cb1, rb0, rb1, vb0, vb1,
             sr0, sr1, zbuf, si0, si1, sg0, sg1, ss0, ss1):
        core = lax.axis_index("c")
        sub = lax.axis_index("s")
        tile_base = sub * per_tile
        H = (h0, h1)
        G = (g0, g1)
        SRB = (sr0, sr1)
        SS = (ss0, ss1)
        CBF = (cb0, cb1)
        RBF = (rb0, rb1)
        VBF = (vb0, vb1)
        SI = (si0, si1)
        SG = (sg0, sg1)

        def idx_start(i, p):
            base = tile_base + i * K
            d1 = pltpu.async_copy(cols_hbm.at[pl.ds(base, K)], CBF[p], SI[p])
            d2 = pltpu.async_copy(rows_hbm.at[pl.ds(base, K)], RBF[p], SI[p])
            d3 = pltpu.async_copy(vals_hbm.at[pl.ds(base, K)], VBF[p], SI[p])
            return d1, d2, d3

        def idx_wait(p):
            for d in idx_make(p):
                d.wait()

        def idx_make(p):
            return (
                pltpu.make_async_copy(
                    cols_hbm.at[pl.ds(0, K)], CBF[p], SI[p]),
                pltpu.make_async_copy(rows_hbm.at[pl.ds(0, K)], RBF[p], SI[p]),
                pltpu.make_async_copy(vals_hbm.at[pl.ds(0, K)], VBF[p], SI[p]),
            )

        def gather_start(p):
            return pltpu.async_copy(tslab.at[CBF[p]], H[p], SG[p])

        def gather_wait(p):
            pltpu.make_async_copy(tslab.at[CBF[p]], H[p], SG[p]).wait()

        def scat_wait(p):
            pltpu.make_async_copy(
                G[p], acc.at[SRB[p]], SS[p]).wait()

        def scale(p):
            hbuf = H[p]
            gbuf = G[p]
            vbuf = VBF[p]
            for j0 in range(0, K, LANES):
                v16 = vbuf[pl.ds(j0, LANES)]
                for l in range(LANES):
                    bc = _lane_bcast(v16, l)
                    j = j0 + l
                    for q in range(CB // 32):
                        h32 = hbuf[j, pl.ds(q * 32, 32)]
                        fa, fb = plsc.unpack(
                            h32, format=plsc.PackFormat.INTERLEAVED)
                        gbuf[j, pl.ds(q * 32, LANES)] = fa * bc
                        gbuf[j, pl.ds(q * 32 + LANES, LANES)] = fb * bc

        zv = jnp.zeros((LANES,), jnp.float32)
        for r in range(ZR):
            for q in range(CB // LANES):
                zbuf[r, pl.ds(q * LANES, LANES)] = zv

        def chunk_body(cc, carry):
            chunk = core * (NCHUNK // NC) + cc

            pltpu.sync_copy(
                t_hbm.at[pl.ds(chunk * T_PAD + sub * SLAB_PT, SLAB_PT)],
                tslab.at[pl.ds(sub * SLAB_PT, SLAB_PT)])
            for p in range(ROWS_PER_TILE // ZR):
                pltpu.sync_copy(
                    zbuf, acc.at[pl.ds(sub * ROWS_PER_TILE + p * ZR, ZR)])
            plsc.subcore_barrier()

            idx_start(0, 0)
            idx_start(1, 1)
            idx_wait(0)
            gather_start(0)

            def body(q, carry):
                for r in range(2):
                    i = 2 * q + r
                    cur, oth = r, 1 - r
                    idx_wait(oth)
                    gather_start(oth)
                    gather_wait(cur)
                    pl.when(q >= 1)(functools.partial(scat_wait, cur))
                    scale(cur)
                    for u in range(K // LANES):
                        sl = pl.ds(u * LANES, LANES)
                        SRB[cur][sl] = RBF[cur][sl]
                    pltpu.async_copy(
                        G[cur], acc.at[SRB[cur]], SS[cur], add=True)
                    idx_start(i + 2, cur)
                return carry

            lax.fori_loop(0, n_blk // 2, body, 0)
            scat_wait(0)
            scat_wait(1)
            gather_wait(n_blk % 2)
            idx_wait((n_blk + 1) % 2)
            plsc.subcore_barrier()

            pltpu.sync_copy(
                acc.at[pl.ds(sub * ROWS_PER_TILE, ROWS_PER_TILE)],
                y_hbm.at[chunk, pl.ds(sub * ROWS_PER_TILE, ROWS_PER_TILE)])
            plsc.subcore_barrier()
            return carry

        lax.fori_loop(0, NCHUNK // NC, chunk_body, 0)

    return spmm


def _erf_body(y_ref, b_ref, o_ref):
    z = lax.erf(y_ref[...] + b_ref[0, 0, :][None, :, None])
    o_ref[...] = jnp.transpose(z, (0, 2, 1))


RB = 512


def kernel(state, x, vals_res, rows_res, cols_res, bias_res,
           vals_in, rows_in, cols_in, bias_in):
    nnz_res = vals_res.shape[0]
    nnz_in = vals_in.shape[0]
    nnz_tot = nnz_res + nnz_in
    n_blk = -(-nnz_tot // (NS * K))
    n_blk += n_blk % 2
    npad = n_blk * K * NS + 2 * K - nnz_tot

    t = jnp.concatenate(
        [state.T, x.T, jnp.ones((1, BATCH), jnp.float32),
         jnp.zeros((T_PAD - T_ROWS, BATCH), jnp.float32)], axis=0)
    t = t[:, _COL_PERM].astype(jnp.bfloat16)
    t4 = t.reshape(T_PAD, NCHUNK, CB).transpose(1, 0, 2)
    t4 = t4.reshape(NCHUNK * T_PAD, CB)

    i32 = jnp.int32
    cols = jnp.concatenate([
        cols_res.astype(i32), cols_in.astype(i32) + N_RES,
        jnp.zeros((npad,), i32)])
    rows = jnp.concatenate([
        rows_res.astype(i32), rows_in.astype(i32), jnp.zeros((npad,), i32)])
    vals = jnp.concatenate([
        vals_res, vals_in, jnp.zeros((npad,), jnp.float32)])

    y4 = _make_spmm(n_blk)(t4, cols, rows, vals)

    bias2 = (bias_res + bias_in).reshape(N_RES // RB, 1, RB)
    out3 = pl.pallas_call(
        _erf_body,
        grid=(N_RES // RB,),
        in_specs=[pl.BlockSpec((NCHUNK, RB, CB), lambda i: (0, i, 0)),
                  pl.BlockSpec((1, 1, RB), lambda i: (i, 0, 0))],
        out_specs=pl.BlockSpec((NCHUNK, CB, RB), lambda i: (0, 0, i)),
        out_shape=jax.ShapeDtypeStruct((NCHUNK, CB, N_RES), jnp.float32),
    )(y4, bias2)
    return out3.reshape(BATCH, N_RES)
